# baseline (speedup 1.0000x reference)
import functools

import jax
import jax.numpy as jnp
from jax import lax
from jax.experimental import pallas as pl
from jax.experimental.pallas import tpu as pltpu
from jax.experimental.pallas import tpu_sc as plsc

_VOCAB = 100000
_DIM = 128
_HALF = _DIM // 2
_EPS = 1e-12

_NORM_BLOCK = 2000

_NC = 2
_NS = 16
_NW = _NC * _NS
_G = 128
_NBUF = 4

_PHASES = 2
_WIDEN_BLOCK = 1600


def _norm_body(t_ref, w_ref, o_ref):
    x = t_ref[...]
    ms = jnp.mean(x * x, axis=-1, keepdims=True)
    y = x * lax.rsqrt(ms + _EPS) * w_ref[...]
    a = lax.bitcast_convert_type(
        y[:, :_HALF].astype(jnp.bfloat16), jnp.uint16).astype(jnp.uint32)
    b = lax.bitcast_convert_type(
        y[:, _HALF:].astype(jnp.bfloat16), jnp.uint16).astype(jnp.uint32)
    o_ref[...] = lax.bitcast_convert_type(a | (b << 16), jnp.int32)


def _normalize_table(table, norm_weight):
    return pl.pallas_call(
        _norm_body,
        grid=(_VOCAB // _NORM_BLOCK,),
        in_specs=[
            pl.BlockSpec((_NORM_BLOCK, _DIM), lambda i: (i, 0)),
            pl.BlockSpec((1, _DIM), lambda i: (0, 0)),
        ],
        out_specs=pl.BlockSpec((_NORM_BLOCK, _HALF), lambda i: (i, 0)),
        out_shape=jax.ShapeDtypeStruct((_VOCAB, _HALF), jnp.int32),
    )(table, norm_weight.reshape(1, _DIM))


def _make_gather(n_ids):
    b_per_w = n_ids // _NW
    n_groups = b_per_w // _G
    assert n_ids == b_per_w * _NW and b_per_w == n_groups * _G
    assert n_groups % _NBUF == 0
    mesh = plsc.VectorSubcoreMesh(
        core_axis_name="c", subcore_axis_name="s",
        num_cores=_NC, num_subcores=_NS,
    )

    @functools.partial(
        pl.kernel,
        out_type=jax.ShapeDtypeStruct((n_ids, _HALF), jnp.int32),
        mesh=mesh,
        scratch_types=[
            pltpu.VMEM((b_per_w,), jnp.int32),
            pltpu.VMEM((_NBUF, _G, _HALF), jnp.int32),
            pltpu.SemaphoreType.DMA((_NBUF,)),
            pltpu.SemaphoreType.DMA((_NBUF,)),
        ],
        compiler_params=pltpu.CompilerParams(use_tc_tiling_on_sc=False),
    )
    def gather_kernel(tab_hbm, ids_hbm, out_hbm, idx_v, rows_v, gsem, wsem):
        wid = lax.axis_index("s") * _NC + lax.axis_index("c")
        base = wid * b_per_w
        pltpu.sync_copy(ids_hbm.at[pl.ds(base, b_per_w)], idx_v)

        def start_gather(b, g):
            pltpu.async_copy(
                tab_hbm.at[idx_v.at[pl.ds(g * _G, _G)]],
                rows_v.at[b], gsem.at[b],
            )

        def wait_gather(b, g):
            pltpu.make_async_copy(
                tab_hbm.at[idx_v.at[pl.ds(g * _G, _G)]],
                rows_v.at[b], gsem.at[b],
            ).wait()

        def start_write(b, g):
            pltpu.async_copy(
                rows_v.at[b], out_hbm.at[pl.ds(base + g * _G, _G)], wsem.at[b]
            )

        def wait_write(b, g):
            pltpu.make_async_copy(
                rows_v.at[b], out_hbm.at[pl.ds(base + g * _G, _G)], wsem.at[b]
            ).wait()

        for b in range(_NBUF):
            start_gather(b, b)

        def outer(it, carry):
            g0 = it * _NBUF
            for b in range(_NBUF):
                g = g0 + b
                wait_gather(b, g)
                start_write(b, g)
                wait_write(b, g)
                start_gather(b, g + _NBUF)
            return carry

        lax.fori_loop(0, n_groups // _NBUF - 1, outer, 0)

        for b in range(_NBUF):
            g = n_groups - _NBUF + b
            wait_gather(b, g)
            start_write(b, g)
        for b in range(_NBUF):
            g = n_groups - _NBUF + b
            wait_write(b, g)

    return gather_kernel


def _widen_body(pk_ref, o_ref):
    x = pk_ref[...]
    o_ref[:, :_HALF] = x << 16
    o_ref[:, _HALF:] = x & jnp.int32(-65536)


def _widen_body_aliased(buf_ref, pk_ref, o_ref):
    del buf_ref
    x = pk_ref[...]
    o_ref[:, :_HALF] = x << 16
    o_ref[:, _HALF:] = x & jnp.int32(-65536)


def _widen(packed, phase, n_total, buf):
    n_phase = packed.shape[0]
    steps = n_phase // _WIDEN_BLOCK
    assert n_phase == steps * _WIDEN_BLOCK
    row0 = phase * steps
    out_spec = pl.BlockSpec((_WIDEN_BLOCK, _DIM), lambda j: (j + row0, 0))
    pk_spec = pl.BlockSpec((_WIDEN_BLOCK, _HALF), lambda j: (j, 0))
    out_shape = jax.ShapeDtypeStruct((n_total, _DIM), jnp.int32)
    if buf is None:
        return pl.pallas_call(
            _widen_body,
            grid=(steps,),
            in_specs=[pk_spec],
            out_specs=out_spec,
            out_shape=out_shape,
        )(packed)
    return pl.pallas_call(
        _widen_body_aliased,
        grid=(steps,),
        in_specs=[pl.BlockSpec(memory_space=pl.ANY), pk_spec],
        out_specs=out_spec,
        out_shape=out_shape,
        input_output_aliases={0: 0},
    )(buf, packed)


def kernel(input_ids, table, norm_weight):
    b, l = input_ids.shape
    n_ids = b * l
    assert n_ids % _PHASES == 0
    n_phase = n_ids // _PHASES

    normed = _normalize_table(table, norm_weight)
    ids_flat = input_ids.reshape(-1)

    gather = _make_gather(n_phase)
    packed = [
        gather(normed, lax.slice(ids_flat, (p * n_phase,),
                                 ((p + 1) * n_phase,)))
        for p in range(_PHASES)
    ]
    buf = None
    for p in range(_PHASES):
        buf = _widen(packed[p], p, n_ids, buf)
    return lax.bitcast_convert_type(buf, jnp.float32).reshape(b, l, _DIM)


# speedup vs baseline: 3.4114x; 3.4114x over previous
import functools

import jax
import jax.numpy as jnp
from jax import lax
from jax.experimental import pallas as pl
from jax.experimental.pallas import tpu as pltpu
from jax.experimental.pallas import tpu_sc as plsc

_VOCAB = 100000
_DIM = 128
_EPS = 1e-12

_NORM_BLOCK = 2000

_NC = 2
_NS = 16
_NW = _NC * _NS
_G = 128


def _norm_body(t_ref, w_ref, o_ref):
    x = t_ref[...]
    ms = jnp.mean(x * x, axis=-1, keepdims=True)
    o_ref[...] = x * lax.rsqrt(ms + _EPS) * w_ref[...]


def _normalize_table(table, norm_weight):
    return pl.pallas_call(
        _norm_body,
        grid=(_VOCAB // _NORM_BLOCK,),
        in_specs=[
            pl.BlockSpec((_NORM_BLOCK, _DIM), lambda i: (i, 0)),
            pl.BlockSpec((1, _DIM), lambda i: (0, 0)),
        ],
        out_specs=pl.BlockSpec((_NORM_BLOCK, _DIM), lambda i: (i, 0)),
        out_shape=jax.ShapeDtypeStruct((_VOCAB, _DIM), jnp.float32),
    )(table, norm_weight.reshape(1, _DIM))


_NBUF = 4


def _make_gather(n_ids):
    assert n_ids % (_NW * _G * _NBUF) == 0
    b_per_w = n_ids // _NW
    n_groups = b_per_w // _G
    mesh = plsc.VectorSubcoreMesh(
        core_axis_name="c", subcore_axis_name="s",
        num_cores=_NC, num_subcores=_NS,
    )

    @functools.partial(
        pl.kernel,
        out_type=jax.ShapeDtypeStruct((n_ids, _DIM), jnp.float32),
        mesh=mesh,
        scratch_types=[
            pltpu.VMEM((b_per_w,), jnp.int32),
            pltpu.VMEM((_NBUF, _G, _DIM), jnp.float32),
            pltpu.SemaphoreType.DMA((_NBUF,)),
            pltpu.SemaphoreType.DMA((_NBUF,)),
        ],
    )
    def gather_kernel(tab_hbm, ids_hbm, out_hbm, idx_v, rows_v, gsem, wsem):
        wid = lax.axis_index("s") * _NC + lax.axis_index("c")
        base = wid * b_per_w
        pltpu.sync_copy(ids_hbm.at[pl.ds(base, b_per_w)], idx_v)

        def start_gather(b, g):
            pltpu.async_copy(
                tab_hbm.at[idx_v.at[pl.ds(g * _G, _G)]],
                rows_v.at[b], gsem.at[b],
            )

        def wait_gather(b, g):
            pltpu.make_async_copy(
                tab_hbm.at[idx_v.at[pl.ds(g * _G, _G)]],
                rows_v.at[b], gsem.at[b],
            ).wait()

        def start_write(b, g):
            pltpu.async_copy(
                rows_v.at[b], out_hbm.at[pl.ds(base + g * _G, _G)], wsem.at[b]
            )

        def wait_write(b, g):
            pltpu.make_async_copy(
                rows_v.at[b], out_hbm.at[pl.ds(base + g * _G, _G)], wsem.at[b]
            ).wait()

        for b in range(_NBUF):
            start_gather(b, b)

        def outer(it, carry):
            g0 = it * _NBUF
            for b in range(_NBUF):
                g = g0 + b
                wait_gather(b, g)
                start_write(b, g)
                wait_write(b, g)
                start_gather(b, g + _NBUF)
            return carry

        lax.fori_loop(0, n_groups // _NBUF - 1, outer, 0)

        for b in range(_NBUF):
            g = n_groups - _NBUF + b
            wait_gather(b, g)
            start_write(b, g)
        for b in range(_NBUF):
            g = n_groups - _NBUF + b
            wait_write(b, g)

    return gather_kernel


def kernel(input_ids, table, norm_weight):
    b, l = input_ids.shape
    normed = _normalize_table(table, norm_weight)
    ids_flat = input_ids.reshape(-1)
    out = _make_gather(ids_flat.size)(normed, ids_flat)
    return out.reshape(b, l, _DIM)
